# baseline (speedup 1.0000x reference)
import functools

import jax
import jax.numpy as jnp
from jax import lax
from jax.experimental import pallas as pl
from jax.experimental.pallas import tpu as pltpu
from jax.experimental.pallas import tpu_sc as plsc

N_USERS = 5000
N_ITEMS = 5000
N_NODES = N_USERS + N_ITEMS
D = 128
E = 320000

NC = 2
NS = 16
DH = D // NC
ET = E // NS
C = 80
NCH = ET // C
RPT = 624
ZR = 80
NPT = N_NODES // NS
NB = 25


def _lane_bcast(v, j):
    idx = jnp.full((16, 1), j, jnp.int32)
    return lax.gather(
        v, idx,
        lax.GatherDimensionNumbers(offset_dims=(), collapsed_slice_dims=(0,),
                                   start_index_map=(0,)),
        (1,), mode=lax.GatherScatterMode.PROMISE_IN_BOUNDS)


def _hsum16(v):
    lanes = lax.iota(jnp.int32, 16)
    for s in (1, 2, 4, 8):
        idx = (lanes ^ s).reshape(16, 1)
        v = v + lax.gather(
            v, idx,
            lax.GatherDimensionNumbers(offset_dims=(),
                                       collapsed_slice_dims=(0,),
                                       start_index_map=(0,)),
            (1,), mode=lax.GatherScatterMode.PROMISE_IN_BOUNDS)
    return v


def _rsqrt16(x):
    y = jnp.full((16,), 16.0, jnp.float32)
    for _ in range(8):
        y = 0.5 * (y + x / y)
    return 1.0 / y


def _sc_forward(state2, xu, xi, dst3, src3, vals3):
    mesh = plsc.VectorSubcoreMesh(core_axis_name="c", subcore_axis_name="s",
                                  num_cores=NC, num_subcores=NS)

    @functools.partial(
        pl.kernel,
        mesh=mesh,
        out_type=(jax.ShapeDtypeStruct((NC, N_USERS, DH), jnp.float32),
                  jax.ShapeDtypeStruct((NC, N_ITEMS, DH), jnp.float32)),
        compiler_params=pltpu.CompilerParams(use_tc_tiling_on_sc=False),
        scratch_types=[
            pltpu.VMEM((NCH, C), jnp.int32),
            pltpu.VMEM((NCH, C), jnp.int32),
            pltpu.VMEM((NCH, C), jnp.float32),
            pltpu.VMEM((C,), jnp.int32),
            pltpu.VMEM((C,), jnp.int32),
            pltpu.VMEM((C, DH), jnp.float32),
            pltpu.VMEM((C, DH), jnp.float32),
            pltpu.VMEM((C, DH), jnp.float32),
            pltpu.VMEM((C, DH), jnp.float32),
            pltpu.VMEM((ZR, DH), jnp.float32),
            pltpu.VMEM((NB, D), jnp.float32),
            pltpu.VMEM((16,), jnp.float32),
            pltpu.VMEM_SHARED((N_NODES, DH), jnp.float32),
            pltpu.VMEM_SHARED((NS, 16), jnp.float32),
            pltpu.SemaphoreType.DMA,
            pltpu.SemaphoreType.DMA,
            pltpu.SemaphoreType.DMA,
            pltpu.SemaphoreType.DMA,
            pltpu.SemaphoreType.DMA,
            pltpu.SemaphoreType.DMA,
        ],
    )
    def k(state_hbm, xu_hbm, xi_hbm, dst_hbm, src_hbm, val_hbm, yu_hbm, yi_hbm,
          src_v, dst_v, val_v, gidx_a, gidx_b, rows_a, rows_b, sc_a, sc_b,
          zero_v, norm_v, nsc_v, acc, nsh,
          sem_e, sem_a, sem_b, sem_sa, sem_sb, sem_z):
        cid = lax.axis_index("c")
        sid = lax.axis_index("s")

        cp1 = pltpu.async_copy(src_hbm.at[sid], src_v, sem_e)
        cp2 = pltpu.async_copy(dst_hbm.at[sid], dst_v, sem_e)
        cp3 = pltpu.async_copy(val_hbm.at[sid], val_v, sem_e)

        zeros16 = jnp.zeros((16,), jnp.float32)

        def zfill(i, _):
            zero_v[i // (DH // 16), pl.ds((i % (DH // 16)) * 16, 16)] = zeros16
            return 0
        lax.fori_loop(0, ZR * (DH // 16), zfill, 0)
        zcps = [pltpu.async_copy(zero_v, acc.at[pl.ds(sid * RPT + z * ZR, ZR)],
                                 sem_z) for z in range(8)]

        def nblock(b, m):
            base = sid * NPT + b * NB

            @pl.when(sid < 8)
            def _():
                pltpu.sync_copy(xu_hbm.at[pl.ds(base, NB)], norm_v)

            @pl.when(sid >= 8)
            def _():
                pltpu.sync_copy(xi_hbm.at[pl.ds(base - N_USERS, NB)], norm_v)

            def node(i, mm):
                v = jnp.zeros((16,), jnp.float32)
                for kk in range(D // 16):
                    x = norm_v[i, pl.ds(kk * 16, 16)]
                    v = v + x * x
                return jnp.maximum(mm, _hsum16(v))
            return lax.fori_loop(0, NB, node, m)
        maxsq = lax.fori_loop(0, NPT // NB, nblock,
                              jnp.zeros((16,), jnp.float32))

        nsc_v[pl.ds(0, 16)] = maxsq
        pltpu.sync_copy(nsc_v, nsh.at[sid])
        plsc.subcore_barrier()
        m16 = jnp.zeros((16,), jnp.float32)
        for t in range(NS):
            pltpu.sync_copy(nsh.at[t], nsc_v)
            m16 = jnp.maximum(m16, nsc_v[pl.ds(0, 16)])
        inv = _rsqrt16(m16)

        cp1.wait()
        cp2.wait()
        cp3.wait()
        for zc in zcps:
            zc.wait()
        plsc.subcore_barrier()

        def gather(chunk, gidx, rows, sem):
            for g in range(C // 16):
                sl = pl.ds(g * 16, 16)
                gidx[sl] = (src_v[chunk, sl] << 1) + cid
            return pltpu.async_copy(state_hbm.at[gidx], rows, sem)

        def wait_gather(gidx, rows, sem):
            pltpu.make_async_copy(state_hbm.at[gidx], rows, sem).wait()

        def scale(chunk, rows, dst):
            for g in range(C // 16):
                v16 = val_v[chunk, pl.ds(g * 16, 16)] * inv
                for j in range(16):
                    vj = _lane_bcast(v16, j)
                    for kk in range(DH // 16):
                        sl = pl.ds(kk * 16, 16)
                        dst[g * 16 + j, sl] = rows[g * 16 + j, sl] * vj

        def scatter(chunk, buf, sem):
            pltpu.async_copy(buf, acc.at[dst_v.at[chunk]], sem, add=True)

        def wait_scatter(buf, sem):
            pltpu.make_async_copy(buf, acc.at[dst_v.at[0]], sem).wait()

        gather(0, gidx_a, rows_a, sem_a)

        def pair(i, _):
            a = 2 * i
            gather(a + 1, gidx_b, rows_b, sem_b)
            wait_gather(gidx_a, rows_a, sem_a)

            @pl.when(i > 0)
            def _():
                wait_scatter(sc_a, sem_sa)
            scale(a, rows_a, sc_a)
            scatter(a, sc_a, sem_sa)

            @pl.when(i < NCH // 2 - 1)
            def _():
                gather(a + 2, gidx_a, rows_a, sem_a)
            wait_gather(gidx_b, rows_b, sem_b)

            @pl.when(i > 0)
            def _():
                wait_scatter(sc_b, sem_sb)
            scale(a + 1, rows_b, sc_b)
            scatter(a + 1, sc_b, sem_sb)
            return 0
        lax.fori_loop(0, NCH // 2, pair, 0)
        wait_scatter(sc_a, sem_sa)
        wait_scatter(sc_b, sem_sb)
        plsc.subcore_barrier()

        @pl.when(sid < 8)
        def _():
            pltpu.sync_copy(acc.at[pl.ds(sid * RPT, RPT)],
                            yu_hbm.at[cid, pl.ds(sid * RPT, RPT)])

        @pl.when(sid == 8)
        def _():
            pltpu.sync_copy(acc.at[pl.ds(8 * RPT, N_USERS - 8 * RPT)],
                            yu_hbm.at[cid, pl.ds(8 * RPT, N_USERS - 8 * RPT)])
            pltpu.sync_copy(acc.at[pl.ds(N_USERS, 9 * RPT - N_USERS)],
                            yi_hbm.at[cid, pl.ds(0, 9 * RPT - N_USERS)])

        @pl.when(sid > 8)
        def _():
            pltpu.sync_copy(acc.at[pl.ds(sid * RPT, RPT)],
                            yi_hbm.at[cid, pl.ds(sid * RPT - N_USERS, RPT)])

        @pl.when(sid == NS - 1)
        def _():
            pltpu.sync_copy(
                acc.at[pl.ds(NS * RPT, N_NODES - NS * RPT)],
                yi_hbm.at[cid, pl.ds(NS * RPT - N_USERS, N_NODES - NS * RPT)])

    return k(state2, xu, xi, dst3, src3, vals3)


def kernel(adj_indices, adj_values, dt, xu, xi, user_states, item_states):
    state = jnp.concatenate([xu, xi], axis=0)
    state2 = state.reshape(2 * N_NODES, DH)
    dst3 = adj_indices[0].reshape(NS, NCH, C)
    src3 = adj_indices[1].reshape(NS, NCH, C)
    val3 = adj_values.reshape(NS, NCH, C)
    yu_p, yi_p = _sc_forward(state2, xu, xi, dst3, src3, val3)
    yu = yu_p.transpose(1, 0, 2).reshape(N_USERS, D)
    yi = yi_p.transpose(1, 0, 2).reshape(N_ITEMS, D)
    return yu, yi
